# baseline (speedup 1.0000x reference)
import functools

import jax
import jax.numpy as jnp
from jax import lax
from jax.experimental import pallas as pl
from jax.experimental.pallas import tpu as pltpu
from jax.experimental.pallas import tpu_sc as plsc

NN = 10000
EE = 320000
DD = 128

NC = 2
NS = 16
NW = NC * NS
K = 64
CPT = 160
E_PAD = NW * CPT * K
ROWS_PAD = E_PAD // K
R = 10112
ZR = R // NS
CH = 128
S = R // CH
HF = 80

_mesh = plsc.VectorSubcoreMesh(core_axis_name="c", subcore_axis_name="s")


def _deg_body(srcp, dstp, ones_h, zeros_h, od_out, id_out,
              acc, src_v, dst_v, ones_v):
    cid = lax.axis_index("c")
    sid = lax.axis_index("s")
    w = cid * NS + sid
    my_rows = pl.ds(sid * ZR, ZR)
    pltpu.sync_copy(zeros_h, acc.at[my_rows])
    pltpu.sync_copy(srcp.at[pl.ds(w * CPT, CPT)], src_v)
    pltpu.sync_copy(dstp.at[pl.ds(w * CPT, CPT)], dst_v)
    pltpu.sync_copy(ones_h, ones_v)
    plsc.subcore_barrier()

    @pl.loop(0, CPT)
    def _(j):
        pltpu.sync_copy(ones_v, acc.at[src_v.at[j]], add=True)

    plsc.subcore_barrier()
    pltpu.sync_copy(acc.at[my_rows], od_out.at[cid, my_rows])
    pltpu.sync_copy(zeros_h, acc.at[my_rows])
    plsc.subcore_barrier()

    @pl.loop(0, CPT)
    def _(j):
        pltpu.sync_copy(ones_v, acc.at[dst_v.at[j]], add=True)

    plsc.subcore_barrier()
    pltpu.sync_copy(acc.at[my_rows], id_out.at[cid, my_rows])


_deg_kernel = pl.kernel(
    _deg_body,
    out_type=(jax.ShapeDtypeStruct((NC, R, DD), jnp.float32),
              jax.ShapeDtypeStruct((NC, R, DD), jnp.float32)),
    mesh=_mesh,
    scratch_types=[
        pltpu.VMEM_SHARED((R, DD), jnp.float32),
        pltpu.VMEM((CPT, K), jnp.int32),
        pltpu.VMEM((CPT, K), jnp.int32),
        pltpu.VMEM((K, DD), jnp.float32),
    ],
)


NB = 2
CTILE = 2 * CPT
PMAX = 112


def _mp_piece(h_tab, srcp, dstp, acc, src_v, dst_v, rows_v, gsem, ssem,
              off, cnt):
    pltpu.sync_copy(srcp.at[pl.ds(off, cnt)], src_v.at[pl.ds(0, cnt)])
    pltpu.sync_copy(dstp.at[pl.ds(off, cnt)], dst_v.at[pl.ds(0, cnt)])

    for b in range(NB):
        pltpu.async_copy(h_tab.at[src_v.at[b]], rows_v.at[b], gsem[b])

    @pl.loop(0, cnt, step=NB)
    def _(j0):
        for b in range(NB):
            pltpu.make_async_copy(
                h_tab.at[pl.ds(0, K)], rows_v.at[b], gsem[b]).wait()
            pltpu.async_copy(rows_v.at[b], acc.at[dst_v.at[j0 + b]],
                             ssem[b], add=True)
        for b in range(NB):
            nxt = j0 + b + NB

            @pl.when(nxt < cnt)
            def _():
                pltpu.make_async_copy(
                    rows_v.at[b], acc.at[pl.ds(0, K)], ssem[b]).wait()
                pltpu.async_copy(h_tab.at[src_v.at[nxt]], rows_v.at[b],
                                 gsem[b])

    for b in range(NB):
        pltpu.make_async_copy(
            rows_v.at[b], acc.at[pl.ds(0, K)], ssem[b]).wait()


def _mp_core(h_tab, srcp, dstp, acc, src_v, dst_v, rows_v, gsem, ssem,
             base, nch):
    done = 0
    while done < nch:
        cnt = min(PMAX, nch - done)
        _mp_piece(h_tab, srcp, dstp, acc, src_v, dst_v, rows_v, gsem, ssem,
                  base + done, cnt)
        done += cnt


def _mp_body(h_tab, srcp, dstp, zeros_h, m_out,
             acc, src_v, dst_v, rows_v, *sems):
    gsem = sems[:NB]
    ssem = sems[NB:]
    cid = lax.axis_index("c")
    sid = lax.axis_index("s")

    @pl.when(cid == 0)
    def _():
        pltpu.sync_copy(zeros_h, acc.at[pl.ds(sid * ZR, ZR)])

    plsc.subcore_barrier()

    @pl.when(cid == 0)
    def _():
        _mp_core(h_tab, srcp, dstp, acc, src_v, dst_v, rows_v, gsem, ssem,
                 sid * CTILE, CTILE)

    plsc.subcore_barrier()

    @pl.when(cid == 0)
    def _():
        pltpu.sync_copy(acc.at[pl.ds(sid * ZR, ZR)],
                        m_out.at[pl.ds(sid * ZR, ZR)])


_mp_kernel = pl.kernel(
    _mp_body,
    out_type=jax.ShapeDtypeStruct((R, DD), jnp.float32),
    mesh=_mesh,
    scratch_types=[
        pltpu.VMEM_SHARED((R, DD), jnp.float32),
        pltpu.VMEM((PMAX, K), jnp.int32),
        pltpu.VMEM((PMAX, K), jnp.int32),
        pltpu.VMEM((NB, K, DD), jnp.float32),
    ] + [pltpu.SemaphoreType.DMA] * (2 * NB),
)


def _prescale_body(feat_ref, od0_ref, od1_ref, out_ref):
    def step(i, carry):
        sl = pl.ds(i * CH, CH)
        deg = od0_ref[sl, 0:1] + od1_ref[sl, 0:1]
        c = lax.rsqrt(jnp.maximum(deg, 1.0))
        out_ref[sl, :] = feat_ref[sl, :] * c
        return carry

    lax.fori_loop(0, S, step, 0)


_prescale = pl.pallas_call(
    _prescale_body,
    out_shape=jax.ShapeDtypeStruct((R, DD), jnp.float32),
)


def _post_body(scale_next, m_ref, id0_ref, id1_ref,
               od0_ref, od1_ref, w_ref, b_ref, g_ref, beta_ref, a_ref,
               out_ref, z_ref):
    W = w_ref[...]
    b = b_ref[...]
    nvalid = float(NN)

    def pass1(i, s):
        sl = pl.ds(i * CH, CH)
        idg = id0_ref[sl, 0:1] + id1_ref[sl, 0:1]
        cd = lax.rsqrt(jnp.maximum(idg, 1.0))
        mc = m_ref[sl, :] * cd
        z = jnp.dot(mc, W, preferred_element_type=jnp.float32) + b
        z_ref[sl, :] = z
        row = i * CH + lax.broadcasted_iota(jnp.int32, (CH, 1), 0)
        zm = jnp.where(row < NN, z, 0.0)
        return s + jnp.sum(zm, axis=0, keepdims=True)

    ssum = lax.fori_loop(0, S, pass1, jnp.zeros((1, DD), jnp.float32))
    mu = ssum / nvalid

    def pass2(i, s):
        sl = pl.ds(i * CH, CH)
        row = i * CH + lax.broadcasted_iota(jnp.int32, (CH, 1), 0)
        d = z_ref[sl, :] - mu
        d = jnp.where(row < NN, d, 0.0)
        return s + jnp.sum(d * d, axis=0, keepdims=True)

    ssq = lax.fori_loop(0, S, pass2, jnp.zeros((1, DD), jnp.float32))
    inv = lax.rsqrt(ssq / nvalid + 1e-5)
    g = g_ref[...]
    beta = beta_ref[...]
    a_row = a_ref[...]

    def pass3(i, carry):
        sl = pl.ds(i * CH, CH)
        y = (z_ref[sl, :] - mu) * inv * g + beta
        y = jnp.where(y >= 0.0, y, a_row * y)
        if scale_next:
            odg = od0_ref[sl, 0:1] + od1_ref[sl, 0:1]
            cs = lax.rsqrt(jnp.maximum(odg, 1.0))
            row = i * CH + lax.broadcasted_iota(jnp.int32, (CH, 1), 0)
            y = jnp.where(row < NN, y * cs, 0.0)
        out_ref[sl, :] = y
        return carry

    lax.fori_loop(0, S, pass3, 0)


def _make_post(scale_next):
    return pl.pallas_call(
        functools.partial(_post_body, scale_next),
        out_shape=jax.ShapeDtypeStruct((R, DD), jnp.float32),
        scratch_shapes=[pltpu.VMEM((R, DD), jnp.float32)],
    )


_post_mid = _make_post(True)
_post_end = _make_post(False)


def kernel(edge_index, feat, W1, b1, g1, beta1, a1, W2, b2, g2, beta2, a2):
    fill = jnp.full((E_PAD - EE,), NN, dtype=jnp.int32)
    srcp = jnp.concatenate([edge_index[0], fill]).reshape(ROWS_PAD, K)
    dstp = jnp.concatenate([edge_index[1], fill]).reshape(ROWS_PAD, K)
    ones128 = jnp.ones((K, DD), jnp.float32)
    zeros128 = jnp.zeros((ZR, DD), jnp.float32)

    od, idg = _deg_kernel(srcp, dstp, ones128, zeros128)
    od0, od1 = od[0, :, :16], od[1, :, :16]
    id0, id1 = idg[0, :, :16], idg[1, :, :16]

    featp = jnp.pad(feat, ((0, R - NN), (0, 0)))
    h1 = _prescale(featp, od0, od1)

    def layer(post, h, W, b, g, beta, a):
        m = _mp_kernel(h, srcp, dstp, zeros128)
        return post(m, id0, id1, od0, od1, W,
                    b.reshape(1, DD), g.reshape(1, DD), beta.reshape(1, DD),
                    jnp.broadcast_to(a.reshape(1, 1), (1, DD)))

    h2 = layer(_post_mid, h1, W1, b1, g1, beta1, a1)
    out = layer(_post_end, h2, W2, b2, g2, beta2, a2)
    return out[:NN]


# speedup vs baseline: 1.2969x; 1.2969x over previous
import functools

import jax
import jax.numpy as jnp
from jax import lax
from jax.experimental import pallas as pl
from jax.experimental.pallas import tpu as pltpu
from jax.experimental.pallas import tpu_sc as plsc

NN = 10000
EE = 320000
DD = 128

NC = 2
NS = 16
NW = NC * NS
K = 64
CPT = 160
E_PAD = NW * CPT * K
ROWS_PAD = E_PAD // K
R = 10112
ZR = R // NS
CH = 128
S = R // CH
HF = 80

_mesh = plsc.VectorSubcoreMesh(core_axis_name="c", subcore_axis_name="s")


def _deg_body(srcp, dstp, ones_h, zeros_h, od_out, id_out,
              acc, src_v, dst_v, ones_v):
    cid = lax.axis_index("c")
    sid = lax.axis_index("s")
    w = cid * NS + sid
    my_rows = pl.ds(sid * ZR, ZR)
    pltpu.sync_copy(zeros_h, acc.at[my_rows])
    pltpu.sync_copy(srcp.at[pl.ds(w * CPT, CPT)], src_v)
    pltpu.sync_copy(dstp.at[pl.ds(w * CPT, CPT)], dst_v)
    pltpu.sync_copy(ones_h, ones_v)
    plsc.subcore_barrier()

    @pl.loop(0, CPT)
    def _(j):
        pltpu.sync_copy(ones_v, acc.at[src_v.at[j]], add=True)

    plsc.subcore_barrier()
    pltpu.sync_copy(acc.at[my_rows], od_out.at[cid, my_rows])
    pltpu.sync_copy(zeros_h, acc.at[my_rows])
    plsc.subcore_barrier()

    @pl.loop(0, CPT)
    def _(j):
        pltpu.sync_copy(ones_v, acc.at[dst_v.at[j]], add=True)

    plsc.subcore_barrier()
    pltpu.sync_copy(acc.at[my_rows], id_out.at[cid, my_rows])


_deg_kernel = pl.kernel(
    _deg_body,
    out_type=(jax.ShapeDtypeStruct((NC, R, DD), jnp.float32),
              jax.ShapeDtypeStruct((NC, R, DD), jnp.float32)),
    mesh=_mesh,
    scratch_types=[
        pltpu.VMEM_SHARED((R, DD), jnp.float32),
        pltpu.VMEM((CPT, K), jnp.int32),
        pltpu.VMEM((CPT, K), jnp.int32),
        pltpu.VMEM((K, DD), jnp.float32),
    ],
)


NB = 2
C0 = 256
C1 = 2 * CPT - C0
PMAX = 112


def _mp_piece(h_tab, srcp, dstp, acc, src_v, dst_v, rows_v, gsem, ssem,
              off, cnt):
    pltpu.sync_copy(srcp.at[pl.ds(off, cnt)], src_v.at[pl.ds(0, cnt)])
    pltpu.sync_copy(dstp.at[pl.ds(off, cnt)], dst_v.at[pl.ds(0, cnt)])

    for b in range(NB):
        pltpu.async_copy(h_tab.at[src_v.at[b]], rows_v.at[b], gsem[b])

    @pl.loop(0, cnt, step=NB)
    def _(j0):
        for b in range(NB):
            pltpu.make_async_copy(
                h_tab.at[pl.ds(0, K)], rows_v.at[b], gsem[b]).wait()
            pltpu.async_copy(rows_v.at[b], acc.at[dst_v.at[j0 + b]],
                             ssem[b], add=True)
        for b in range(NB):
            nxt = j0 + b + NB

            @pl.when(nxt < cnt)
            def _():
                pltpu.make_async_copy(
                    rows_v.at[b], acc.at[pl.ds(0, K)], ssem[b]).wait()
                pltpu.async_copy(h_tab.at[src_v.at[nxt]], rows_v.at[b],
                                 gsem[b])

    for b in range(NB):
        pltpu.make_async_copy(
            rows_v.at[b], acc.at[pl.ds(0, K)], ssem[b]).wait()


def _mp_core(h_tab, srcp, dstp, acc, src_v, dst_v, rows_v, gsem, ssem,
             base, nch):
    done = 0
    while done < nch:
        cnt = min(PMAX, nch - done)
        _mp_piece(h_tab, srcp, dstp, acc, src_v, dst_v, rows_v, gsem, ssem,
                  base + done, cnt)
        done += cnt


def _mp_body(h_tab, srcp, dstp, zeros_h, m_out,
             acc, src_v, dst_v, rows_v, *sems):
    gsem = sems[:NB]
    ssem = sems[NB:]
    cid = lax.axis_index("c")
    sid = lax.axis_index("s")
    pltpu.sync_copy(zeros_h, acc.at[pl.ds(sid * ZR, ZR)])
    plsc.subcore_barrier()

    @pl.when(cid == 0)
    def _():
        _mp_core(h_tab, srcp, dstp, acc, src_v, dst_v, rows_v, gsem, ssem,
                 sid * C0, C0)

    @pl.when(cid == 1)
    def _():
        _mp_core(h_tab, srcp, dstp, acc, src_v, dst_v, rows_v, gsem, ssem,
                 NS * C0 + sid * C1, C1)

    plsc.subcore_barrier()
    pltpu.sync_copy(acc.at[pl.ds(sid * ZR, ZR)],
                    m_out.at[cid, pl.ds(sid * ZR, ZR)])


_mp_kernel = pl.kernel(
    _mp_body,
    out_type=jax.ShapeDtypeStruct((NC, R, DD), jnp.float32),
    mesh=_mesh,
    scratch_types=[
        pltpu.VMEM_SHARED((R, DD), jnp.float32),
        pltpu.VMEM((PMAX, K), jnp.int32),
        pltpu.VMEM((PMAX, K), jnp.int32),
        pltpu.VMEM((NB, K, DD), jnp.float32),
    ] + [pltpu.SemaphoreType.DMA] * (2 * NB),
)


def _prescale_body(feat_ref, od0_ref, od1_ref, out_ref):
    def step(i, carry):
        sl = pl.ds(i * CH, CH)
        deg = od0_ref[sl, 0:1] + od1_ref[sl, 0:1]
        c = lax.rsqrt(jnp.maximum(deg, 1.0))
        out_ref[sl, :] = feat_ref[sl, :] * c
        return carry

    lax.fori_loop(0, S, step, 0)


_prescale = pl.pallas_call(
    _prescale_body,
    out_shape=jax.ShapeDtypeStruct((R, DD), jnp.float32),
)


def _post_body(scale_next, m0_ref, m1_ref, id0_ref, id1_ref,
               od0_ref, od1_ref, w_ref, b_ref, g_ref, beta_ref, a_ref,
               out_ref, z_ref):
    W = w_ref[...]
    b = b_ref[...]
    nvalid = float(NN)

    def pass1(i, s):
        sl = pl.ds(i * CH, CH)
        idg = id0_ref[sl, 0:1] + id1_ref[sl, 0:1]
        cd = lax.rsqrt(jnp.maximum(idg, 1.0))
        mc = (m0_ref[sl, :] + m1_ref[sl, :]) * cd
        z = jnp.dot(mc, W, preferred_element_type=jnp.float32) + b
        z_ref[sl, :] = z
        row = i * CH + lax.broadcasted_iota(jnp.int32, (CH, 1), 0)
        zm = jnp.where(row < NN, z, 0.0)
        return s + jnp.sum(zm, axis=0, keepdims=True)

    ssum = lax.fori_loop(0, S, pass1, jnp.zeros((1, DD), jnp.float32))
    mu = ssum / nvalid

    def pass2(i, s):
        sl = pl.ds(i * CH, CH)
        row = i * CH + lax.broadcasted_iota(jnp.int32, (CH, 1), 0)
        d = z_ref[sl, :] - mu
        d = jnp.where(row < NN, d, 0.0)
        return s + jnp.sum(d * d, axis=0, keepdims=True)

    ssq = lax.fori_loop(0, S, pass2, jnp.zeros((1, DD), jnp.float32))
    inv = lax.rsqrt(ssq / nvalid + 1e-5)
    g = g_ref[...]
    beta = beta_ref[...]
    a_row = a_ref[...]

    def pass3(i, carry):
        sl = pl.ds(i * CH, CH)
        y = (z_ref[sl, :] - mu) * inv * g + beta
        y = jnp.where(y >= 0.0, y, a_row * y)
        if scale_next:
            odg = od0_ref[sl, 0:1] + od1_ref[sl, 0:1]
            cs = lax.rsqrt(jnp.maximum(odg, 1.0))
            row = i * CH + lax.broadcasted_iota(jnp.int32, (CH, 1), 0)
            y = jnp.where(row < NN, y * cs, 0.0)
        out_ref[sl, :] = y
        return carry

    lax.fori_loop(0, S, pass3, 0)


def _make_post(scale_next):
    return pl.pallas_call(
        functools.partial(_post_body, scale_next),
        out_shape=jax.ShapeDtypeStruct((R, DD), jnp.float32),
        scratch_shapes=[pltpu.VMEM((R, DD), jnp.float32)],
    )


_post_mid = _make_post(True)
_post_end = _make_post(False)


def kernel(edge_index, feat, W1, b1, g1, beta1, a1, W2, b2, g2, beta2, a2):
    fill = jnp.full((E_PAD - EE,), NN, dtype=jnp.int32)
    srcp = jnp.concatenate([edge_index[0], fill]).reshape(ROWS_PAD, K)
    dstp = jnp.concatenate([edge_index[1], fill]).reshape(ROWS_PAD, K)
    ones128 = jnp.ones((K, DD), jnp.float32)
    zeros128 = jnp.zeros((ZR, DD), jnp.float32)

    od, idg = _deg_kernel(srcp, dstp, ones128, zeros128)
    od0, od1 = od[0, :, :16], od[1, :, :16]
    id0, id1 = idg[0, :, :16], idg[1, :, :16]

    featp = jnp.pad(feat, ((0, R - NN), (0, 0)))
    h1 = _prescale(featp, od0, od1)

    def layer(post, h, W, b, g, beta, a):
        m = _mp_kernel(h, srcp, dstp, zeros128)
        return post(m[0], m[1], id0, id1, od0, od1, W,
                    b.reshape(1, DD), g.reshape(1, DD), beta.reshape(1, DD),
                    jnp.broadcast_to(a.reshape(1, 1), (1, DD)))

    h2 = layer(_post_mid, h1, W1, b1, g1, beta1, a1)
    out = layer(_post_end, h2, W2, b2, g2, beta2, a2)
    return out[:NN]


# speedup vs baseline: 1.3982x; 1.0781x over previous
import functools

import jax
import jax.numpy as jnp
from jax import lax
from jax.experimental import pallas as pl
from jax.experimental.pallas import tpu as pltpu
from jax.experimental.pallas import tpu_sc as plsc

NN = 10000
EE = 320000
DD = 128

NC = 2
NS = 16
NW = NC * NS
K = 64
CPT = 160
E_PAD = NW * CPT * K
ROWS_PAD = E_PAD // K
R = 10112
ZR = R // NS
CH = 128
S = R // CH
HF = 80

_mesh = plsc.VectorSubcoreMesh(core_axis_name="c", subcore_axis_name="s")


def _deg_body(srcp, dstp, ones_h, zeros_h, od_out, id_out,
              acc, src_v, dst_v, ones_v):
    cid = lax.axis_index("c")
    sid = lax.axis_index("s")
    w = cid * NS + sid
    my_rows = pl.ds(sid * ZR, ZR)
    pltpu.sync_copy(zeros_h, acc.at[my_rows])
    pltpu.sync_copy(srcp.at[pl.ds(w * CPT, CPT)], src_v)
    pltpu.sync_copy(dstp.at[pl.ds(w * CPT, CPT)], dst_v)
    pltpu.sync_copy(ones_h, ones_v)
    plsc.subcore_barrier()

    @pl.loop(0, CPT)
    def _(j):
        pltpu.sync_copy(ones_v, acc.at[src_v.at[j]], add=True)

    plsc.subcore_barrier()
    pltpu.sync_copy(acc.at[my_rows], od_out.at[cid, my_rows])
    pltpu.sync_copy(zeros_h, acc.at[my_rows])
    plsc.subcore_barrier()

    @pl.loop(0, CPT)
    def _(j):
        pltpu.sync_copy(ones_v, acc.at[dst_v.at[j]], add=True)

    plsc.subcore_barrier()
    pltpu.sync_copy(acc.at[my_rows], id_out.at[cid, my_rows])


_deg_kernel = pl.kernel(
    _deg_body,
    out_type=(jax.ShapeDtypeStruct((NC, R, DD), jnp.float32),
              jax.ShapeDtypeStruct((NC, R, DD), jnp.float32)),
    mesh=_mesh,
    scratch_types=[
        pltpu.VMEM_SHARED((R, DD), jnp.float32),
        pltpu.VMEM((CPT, K), jnp.int32),
        pltpu.VMEM((CPT, K), jnp.int32),
        pltpu.VMEM((K, DD), jnp.float32),
    ],
)


NB = 2
C0 = 288
C1 = 2 * CPT - C0
PMAX = 112


def _mp_piece(h_tab, srcp, dstp, acc, src_v, dst_v, rows_v, gsem, ssem,
              off, cnt):
    pltpu.sync_copy(srcp.at[pl.ds(off, cnt)], src_v.at[pl.ds(0, cnt)])
    pltpu.sync_copy(dstp.at[pl.ds(off, cnt)], dst_v.at[pl.ds(0, cnt)])

    for b in range(NB):
        pltpu.async_copy(h_tab.at[src_v.at[b]], rows_v.at[b], gsem[b])

    @pl.loop(0, cnt, step=NB)
    def _(j0):
        for b in range(NB):
            pltpu.make_async_copy(
                h_tab.at[pl.ds(0, K)], rows_v.at[b], gsem[b]).wait()
            pltpu.async_copy(rows_v.at[b], acc.at[dst_v.at[j0 + b]],
                             ssem[b], add=True)
        for b in range(NB):
            nxt = j0 + b + NB

            @pl.when(nxt < cnt)
            def _():
                pltpu.make_async_copy(
                    rows_v.at[b], acc.at[pl.ds(0, K)], ssem[b]).wait()
                pltpu.async_copy(h_tab.at[src_v.at[nxt]], rows_v.at[b],
                                 gsem[b])

    for b in range(NB):
        pltpu.make_async_copy(
            rows_v.at[b], acc.at[pl.ds(0, K)], ssem[b]).wait()


def _mp_core(h_tab, srcp, dstp, acc, src_v, dst_v, rows_v, gsem, ssem,
             base, nch):
    done = 0
    while done < nch:
        cnt = min(PMAX, nch - done)
        _mp_piece(h_tab, srcp, dstp, acc, src_v, dst_v, rows_v, gsem, ssem,
                  base + done, cnt)
        done += cnt


def _mp_body(h_tab, srcp, dstp, zeros_h, m_out,
             acc, src_v, dst_v, rows_v, *sems):
    gsem = sems[:NB]
    ssem = sems[NB:]
    cid = lax.axis_index("c")
    sid = lax.axis_index("s")
    pltpu.sync_copy(zeros_h, acc.at[pl.ds(sid * ZR, ZR)])
    plsc.subcore_barrier()

    @pl.when(cid == 0)
    def _():
        _mp_core(h_tab, srcp, dstp, acc, src_v, dst_v, rows_v, gsem, ssem,
                 sid * C0, C0)

    @pl.when(cid == 1)
    def _():
        _mp_core(h_tab, srcp, dstp, acc, src_v, dst_v, rows_v, gsem, ssem,
                 NS * C0 + sid * C1, C1)

    plsc.subcore_barrier()
    pltpu.sync_copy(acc.at[pl.ds(sid * ZR, ZR)],
                    m_out.at[cid, pl.ds(sid * ZR, ZR)])


_mp_kernel = pl.kernel(
    _mp_body,
    out_type=jax.ShapeDtypeStruct((NC, R, DD), jnp.float32),
    mesh=_mesh,
    scratch_types=[
        pltpu.VMEM_SHARED((R, DD), jnp.float32),
        pltpu.VMEM((PMAX, K), jnp.int32),
        pltpu.VMEM((PMAX, K), jnp.int32),
        pltpu.VMEM((NB, K, DD), jnp.float32),
    ] + [pltpu.SemaphoreType.DMA] * (2 * NB),
)


def _prescale_body(feat_ref, od0_ref, od1_ref, out_ref):
    def step(i, carry):
        sl = pl.ds(i * CH, CH)
        deg = od0_ref[sl, 0:1] + od1_ref[sl, 0:1]
        c = lax.rsqrt(jnp.maximum(deg, 1.0))
        out_ref[sl, :] = feat_ref[sl, :] * c
        return carry

    lax.fori_loop(0, S, step, 0)


_prescale = pl.pallas_call(
    _prescale_body,
    out_shape=jax.ShapeDtypeStruct((R, DD), jnp.float32),
)


def _post_body(scale_next, m0_ref, m1_ref, id0_ref, id1_ref,
               od0_ref, od1_ref, w_ref, b_ref, g_ref, beta_ref, a_ref,
               out_ref, z_ref):
    W = w_ref[...]
    b = b_ref[...]
    nvalid = float(NN)

    def pass1(i, s):
        sl = pl.ds(i * CH, CH)
        idg = id0_ref[sl, 0:1] + id1_ref[sl, 0:1]
        cd = lax.rsqrt(jnp.maximum(idg, 1.0))
        mc = (m0_ref[sl, :] + m1_ref[sl, :]) * cd
        z = jnp.dot(mc, W, preferred_element_type=jnp.float32) + b
        z_ref[sl, :] = z
        row = i * CH + lax.broadcasted_iota(jnp.int32, (CH, 1), 0)
        zm = jnp.where(row < NN, z, 0.0)
        return s + jnp.sum(zm, axis=0, keepdims=True)

    ssum = lax.fori_loop(0, S, pass1, jnp.zeros((1, DD), jnp.float32))
    mu = ssum / nvalid

    def pass2(i, s):
        sl = pl.ds(i * CH, CH)
        row = i * CH + lax.broadcasted_iota(jnp.int32, (CH, 1), 0)
        d = z_ref[sl, :] - mu
        d = jnp.where(row < NN, d, 0.0)
        return s + jnp.sum(d * d, axis=0, keepdims=True)

    ssq = lax.fori_loop(0, S, pass2, jnp.zeros((1, DD), jnp.float32))
    inv = lax.rsqrt(ssq / nvalid + 1e-5)
    g = g_ref[...]
    beta = beta_ref[...]
    a_row = a_ref[...]

    def pass3(i, carry):
        sl = pl.ds(i * CH, CH)
        y = (z_ref[sl, :] - mu) * inv * g + beta
        y = jnp.where(y >= 0.0, y, a_row * y)
        if scale_next:
            odg = od0_ref[sl, 0:1] + od1_ref[sl, 0:1]
            cs = lax.rsqrt(jnp.maximum(odg, 1.0))
            row = i * CH + lax.broadcasted_iota(jnp.int32, (CH, 1), 0)
            y = jnp.where(row < NN, y * cs, 0.0)
        out_ref[sl, :] = y
        return carry

    lax.fori_loop(0, S, pass3, 0)


def _make_post(scale_next):
    return pl.pallas_call(
        functools.partial(_post_body, scale_next),
        out_shape=jax.ShapeDtypeStruct((R, DD), jnp.float32),
        scratch_shapes=[pltpu.VMEM((R, DD), jnp.float32)],
    )


_post_mid = _make_post(True)
_post_end = _make_post(False)


def kernel(edge_index, feat, W1, b1, g1, beta1, a1, W2, b2, g2, beta2, a2):
    fill = jnp.full((E_PAD - EE,), NN, dtype=jnp.int32)
    srcp = jnp.concatenate([edge_index[0], fill]).reshape(ROWS_PAD, K)
    dstp = jnp.concatenate([edge_index[1], fill]).reshape(ROWS_PAD, K)
    ones128 = jnp.ones((K, DD), jnp.float32)
    zeros128 = jnp.zeros((ZR, DD), jnp.float32)

    od, idg = _deg_kernel(srcp, dstp, ones128, zeros128)
    od0, od1 = od[0, :, :16], od[1, :, :16]
    id0, id1 = idg[0, :, :16], idg[1, :, :16]

    featp = jnp.pad(feat, ((0, R - NN), (0, 0)))
    h1 = _prescale(featp, od0, od1)

    def layer(post, h, W, b, g, beta, a):
        m = _mp_kernel(h, srcp, dstp, zeros128)
        return post(m[0], m[1], id0, id1, od0, od1, W,
                    b.reshape(1, DD), g.reshape(1, DD), beta.reshape(1, DD),
                    jnp.broadcast_to(a.reshape(1, 1), (1, DD)))

    h2 = layer(_post_mid, h1, W1, b1, g1, beta1, a1)
    out = layer(_post_end, h2, W2, b2, g2, beta2, a2)
    return out[:NN]
